# baseline (speedup 1.0000x reference)
import functools

import jax
import jax.numpy as jnp
from jax import lax
from jax.experimental import pallas as pl
from jax.experimental.pallas import tpu as pltpu
from jax.experimental.pallas import tpu_sc as plsc

B = 16384
HIDDEN = 128
HALF = 64
W2V = 10
NPOS = 32
NC = 2
NS = 16
NW = NC * NS
CHUNK = B // NW
HCHUNK = CHUNK // 2
L = 16


def _fused_rows(pv_ref, t0, ptab_ref, raw_ref, col0, w_ref, b_ref, out_ref):
    wvals = [[w_ref[k, L * j:L * (j + 1)] for j in range(HALF // L)]
             for k in range(W2V)]
    bvals = [b_ref[L * j:L * (j + 1)] for j in range(HALF // L)]
    iota = jnp.arange(L, dtype=jnp.int32)

    @plsc.parallel_loop(0, HCHUNK, 1, unroll=2)
    def body(t):
        idx_t = jnp.full((L,), t, dtype=jnp.int32)
        p_t = plsc.load_gather(pv_ref, [idx_t + t0])
        for j in range(HALF // L):
            out_ref[t, L * j:L * (j + 1)] = plsc.load_gather(
                ptab_ref, [p_t, iota + L * j])
        accs = list(bvals)
        for k in range(W2V):
            idx_k = jnp.full((L,), col0 + k, dtype=jnp.int32)
            rk = plsc.load_gather(raw_ref, [idx_t, idx_k])
            accs = [a + rk * wvals[k][j] for j, a in enumerate(accs)]
        for j in range(HALF // L):
            out_ref[t, HALF + L * j:HALF + L * (j + 1)] = accs[j]


@functools.partial(
    pl.kernel,
    out_type=(
        jax.ShapeDtypeStruct((B, HIDDEN), jnp.float32),
        jax.ShapeDtypeStruct((B, HIDDEN), jnp.float32),
    ),
    mesh=plsc.VectorSubcoreMesh(core_axis_name="c", subcore_axis_name="s",
                                num_cores=NC, num_subcores=NS),
    compiler_params=pltpu.CompilerParams(use_tc_tiling_on_sc=False,
                                         needs_layout_passes=False),
    scratch_types=[
        pltpu.VMEM((CHUNK,), jnp.int32),
        pltpu.VMEM((CHUNK,), jnp.int32),
        pltpu.VMEM((CHUNK,), jnp.int32),
        pltpu.VMEM((NPOS, HALF), jnp.float32),
        pltpu.VMEM((NPOS, HALF), jnp.float32),
        pltpu.VMEM((HCHUNK, HIDDEN), jnp.float32),
        pltpu.VMEM((HCHUNK, HIDDEN), jnp.float32),
        pltpu.VMEM((HCHUNK, HIDDEN), jnp.float32),
        pltpu.VMEM((W2V, HALF), jnp.float32),
        pltpu.VMEM((W2V, HALF), jnp.float32),
        pltpu.VMEM((HALF,), jnp.float32),
        pltpu.VMEM((HALF,), jnp.float32),
        pltpu.SemaphoreType.DMA,
        pltpu.SemaphoreType.DMA,
        pltpu.SemaphoreType.DMA,
    ],
)
def _mixed_context_sc(x_hbm, pos_table_hbm, idx2ctx_hbm, pos_c_hbm,
                      pos_h_hbm, w2v_hbm, c_w_hbm, c_b_hbm,
                      h_w_hbm, h_b_hbm, out_c_hbm, out_h_hbm,
                      xv, pv, cv, ptabc, ptabh, raw0, raw1, outb,
                      wcv, whv, bcv, bhv, s0, s1, s2):
    wid = lax.axis_index("s") * NC + lax.axis_index("c")
    base = wid * CHUNK

    pltpu.sync_copy(x_hbm.at[pl.ds(base, CHUNK)], xv)
    hp = pltpu.async_copy(pos_table_hbm.at[xv], pv, s0)
    hc = pltpu.async_copy(idx2ctx_hbm.at[xv], cv, s1)

    pltpu.sync_copy(pos_c_hbm, ptabc)
    pltpu.sync_copy(pos_h_hbm, ptabh)
    pltpu.sync_copy(c_w_hbm, wcv)
    pltpu.sync_copy(h_w_hbm, whv)
    pltpu.sync_copy(c_b_hbm, bcv)
    pltpu.sync_copy(h_b_hbm, bhv)

    hc.wait()
    hr0 = pltpu.async_copy(w2v_hbm.at[cv.at[pl.ds(0, HCHUNK)]], raw0, s1)
    hr1 = pltpu.async_copy(w2v_hbm.at[cv.at[pl.ds(HCHUNK, HCHUNK)]], raw1, s2)
    hp.wait()

    hr0.wait()
    _fused_rows(pv, 0, ptabc, raw0, 0, wcv, bcv, outb)
    pltpu.sync_copy(outb, out_c_hbm.at[pl.ds(base, HCHUNK)])
    _fused_rows(pv, 0, ptabh, raw0, W2V, whv, bhv, outb)
    pltpu.sync_copy(outb, out_h_hbm.at[pl.ds(base, HCHUNK)])

    hr1.wait()
    _fused_rows(pv, HCHUNK, ptabc, raw1, 0, wcv, bcv, outb)
    pltpu.sync_copy(outb, out_c_hbm.at[pl.ds(base + HCHUNK, HCHUNK)])
    _fused_rows(pv, HCHUNK, ptabh, raw1, W2V, whv, bhv, outb)
    pltpu.sync_copy(outb, out_h_hbm.at[pl.ds(base + HCHUNK, HCHUNK)])


def kernel(x, pos_table, idx2context, pos_c_emb, pos_h_emb, w2v_c_emb,
           w2v_h_emb, c_lin_w, c_lin_b, h_lin_w, h_lin_b):
    w2v = jnp.concatenate(
        [w2v_c_emb, w2v_h_emb,
         jnp.zeros((w2v_c_emb.shape[0], HIDDEN - 2 * W2V), jnp.float32)],
        axis=1)
    out_c, out_h = _mixed_context_sc(
        x, pos_table, idx2context, pos_c_emb, pos_h_emb,
        w2v, c_lin_w, c_lin_b, h_lin_w, h_lin_b)
    return (out_c.reshape(1, B, HIDDEN), out_h.reshape(1, B, HIDDEN))


# speedup vs baseline: 1.3198x; 1.3198x over previous
import functools

import jax
import jax.numpy as jnp
from jax import lax
from jax.experimental import pallas as pl
from jax.experimental.pallas import tpu as pltpu
from jax.experimental.pallas import tpu_sc as plsc

B = 16384
HIDDEN = 128
HALF = 64
W2V = 10
W2VPAD = 32
NPOS = 32
NC = 2
NS = 16
NW = NC * NS
CHUNK = B // NW
HCHUNK = CHUNK // 2
QROWS = 100000 * W2VPAD // HIDDEN
L = 16


def _fused_rows(pv_ref, ov_ref, t0, ptab_ref, raw_ref, col0, w_ref, b_ref,
                out_ref):
    wvals = [[w_ref[k, L * j:L * (j + 1)] for j in range(HALF // L)]
             for k in range(W2V)]
    bvals = [b_ref[L * j:L * (j + 1)] for j in range(HALF // L)]
    iota = jnp.arange(L, dtype=jnp.int32)

    @plsc.parallel_loop(0, HCHUNK, 1, unroll=2)
    def body(t):
        idx_t = jnp.full((L,), t, dtype=jnp.int32)
        p_t = plsc.load_gather(pv_ref, [idx_t + t0])
        for j in range(HALF // L):
            out_ref[t, L * j:L * (j + 1)] = plsc.load_gather(
                ptab_ref, [p_t, iota + L * j])
        o_t = plsc.load_gather(ov_ref, [idx_t + t0])
        accs = list(bvals)
        for k in range(W2V):
            rk = plsc.load_gather(raw_ref, [idx_t, o_t + (col0 + k)])
            accs = [a + rk * wvals[k][j] for j, a in enumerate(accs)]
        for j in range(HALF // L):
            out_ref[t, HALF + L * j:HALF + L * (j + 1)] = accs[j]


@functools.partial(
    pl.kernel,
    out_type=(
        jax.ShapeDtypeStruct((B, HIDDEN), jnp.float32),
        jax.ShapeDtypeStruct((B, HIDDEN), jnp.float32),
    ),
    mesh=plsc.VectorSubcoreMesh(core_axis_name="c", subcore_axis_name="s",
                                num_cores=NC, num_subcores=NS),
    compiler_params=pltpu.CompilerParams(use_tc_tiling_on_sc=False,
                                         needs_layout_passes=False),
    scratch_types=[
        pltpu.VMEM((CHUNK,), jnp.int32),
        pltpu.VMEM((CHUNK,), jnp.int32),
        pltpu.VMEM((CHUNK,), jnp.int32),
        pltpu.VMEM((CHUNK,), jnp.int32),
        pltpu.VMEM((CHUNK,), jnp.int32),
        pltpu.VMEM((NPOS, HALF), jnp.float32),
        pltpu.VMEM((NPOS, HALF), jnp.float32),
        pltpu.VMEM((HCHUNK, HIDDEN), jnp.float32),
        pltpu.VMEM((HCHUNK, HIDDEN), jnp.float32),
        pltpu.VMEM((HCHUNK, HIDDEN), jnp.float32),
        pltpu.VMEM((W2V, HALF), jnp.float32),
        pltpu.VMEM((W2V, HALF), jnp.float32),
        pltpu.VMEM((HALF,), jnp.float32),
        pltpu.VMEM((HALF,), jnp.float32),
        pltpu.SemaphoreType.DMA,
        pltpu.SemaphoreType.DMA,
        pltpu.SemaphoreType.DMA,
    ],
)
def _mixed_context_sc(x_hbm, pos_table_hbm, idx2ctx_hbm, pos_c_hbm,
                      pos_h_hbm, w2v_hbm, c_w_hbm, c_b_hbm,
                      h_w_hbm, h_b_hbm, out_c_hbm, out_h_hbm,
                      xv, pv, cv, gv, ov, ptabc, ptabh, raw0, raw1, outb,
                      wcv, whv, bcv, bhv, s0, s1, s2):
    wid = lax.axis_index("s") * NC + lax.axis_index("c")
    base = wid * CHUNK

    pltpu.sync_copy(x_hbm.at[pl.ds(base, CHUNK)], xv)
    hp = pltpu.async_copy(pos_table_hbm.at[xv], pv, s0)
    hc = pltpu.async_copy(idx2ctx_hbm.at[xv], cv, s1)

    pltpu.sync_copy(pos_c_hbm, ptabc)
    pltpu.sync_copy(pos_h_hbm, ptabh)
    pltpu.sync_copy(c_w_hbm, wcv)
    pltpu.sync_copy(h_w_hbm, whv)
    pltpu.sync_copy(c_b_hbm, bcv)
    pltpu.sync_copy(h_b_hbm, bhv)

    hc.wait()
    for i in range(CHUNK // L):
        cval = cv[pl.ds(i * L, L)]
        gv[pl.ds(i * L, L)] = lax.shift_right_logical(cval, 2)
        ov[pl.ds(i * L, L)] = lax.shift_left(
            jnp.bitwise_and(cval, 3), 5)
    hr0 = pltpu.async_copy(w2v_hbm.at[gv.at[pl.ds(0, HCHUNK)]], raw0, s1)
    hr1 = pltpu.async_copy(w2v_hbm.at[gv.at[pl.ds(HCHUNK, HCHUNK)]], raw1, s2)
    hp.wait()

    hr0.wait()
    _fused_rows(pv, ov, 0, ptabc, raw0, 0, wcv, bcv, outb)
    pltpu.sync_copy(outb, out_c_hbm.at[pl.ds(base, HCHUNK)])
    _fused_rows(pv, ov, 0, ptabh, raw0, W2V, whv, bhv, outb)
    pltpu.sync_copy(outb, out_h_hbm.at[pl.ds(base, HCHUNK)])

    hr1.wait()
    _fused_rows(pv, ov, HCHUNK, ptabc, raw1, 0, wcv, bcv, outb)
    pltpu.sync_copy(outb, out_c_hbm.at[pl.ds(base + HCHUNK, HCHUNK)])
    _fused_rows(pv, ov, HCHUNK, ptabh, raw1, W2V, whv, bhv, outb)
    pltpu.sync_copy(outb, out_h_hbm.at[pl.ds(base + HCHUNK, HCHUNK)])


def kernel(x, pos_table, idx2context, pos_c_emb, pos_h_emb, w2v_c_emb,
           w2v_h_emb, c_lin_w, c_lin_b, h_lin_w, h_lin_b):
    w2v = jnp.concatenate(
        [w2v_c_emb, w2v_h_emb,
         jnp.zeros((w2v_c_emb.shape[0], W2VPAD - 2 * W2V), jnp.float32)],
        axis=1).reshape(QROWS, HIDDEN)
    out_c, out_h = _mixed_context_sc(
        x, pos_table, idx2context, pos_c_emb, pos_h_emb,
        w2v, c_lin_w, c_lin_b, h_lin_w, h_lin_b)
    return (out_c.reshape(1, B, HIDDEN), out_h.reshape(1, B, HIDDEN))
